# baseline (speedup 1.0000x reference)
import functools

import jax
import jax.numpy as jnp
from jax.experimental import pallas as pl
from jax.experimental.pallas import tpu as pltpu

_KH = 5
_PAD = _KH // 2
_INV_ALPHA = 10.0


def _lbp_body(H, W, P, cs_ref, kys_ref, kxs_ref,
              x_hbm, out_ref, cache_ref, sem):
    n = pl.program_id(0)
    f = pl.program_id(1)
    N = pl.num_programs(0)
    _, _, R, L = cache_ref.shape

    def slab_copy(ni):
        return pltpu.make_async_copy(
            x_hbm.at[ni],
            cache_ref.at[ni % 2, :, 0:H, :],
            sem.at[ni % 2],
        )

    @pl.when(f == 0)
    def _():
        @pl.when(n == 0)
        def _():
            cache_ref[...] = jnp.zeros_like(cache_ref)
            slab_copy(0).start()

        @pl.when(n + 1 < N)
        def _():
            slab_copy(n + 1).start()

        slab_copy(n).wait()

    lane = jax.lax.broadcasted_iota(jnp.int32, (H, W), 1)
    acc = None
    for p in range(P):
        idx = f * P + p
        c = cs_ref[idx]
        ky = kys_ref[idx]
        kx = kxs_ref[idx]

        s_full = cache_ref[n % 2, c]
        ctr = s_full[0:H, 0:W]
        s = pltpu.roll(s_full, ((R + _PAD) - ky) % R, 0)
        s = pltpu.roll(s, ((W + _PAD) - kx) % W, 1)
        nb = s[0:H, 0:W]
        nb = jnp.where((lane >= _PAD - kx) & (lane < W + _PAD - kx), nb, 0.0)

        val = float(2 ** p) * jax.nn.sigmoid((nb - ctr) * _INV_ALPHA)
        acc = val if acc is None else acc + val

    out_ref[0, 0] = acc


def kernel(input, kernels, projection_map):
    N, C, H, W = input.shape
    F, P = projection_map.shape

    cs = projection_map.reshape(-1).astype(jnp.int32)
    kys = kernels[..., 0].reshape(-1).astype(jnp.int32)
    kxs = kernels[..., 1].reshape(-1).astype(jnp.int32)

    body = functools.partial(_lbp_body, H, W, P)

    rows = H + 8
    cols = W

    grid_spec = pltpu.PrefetchScalarGridSpec(
        num_scalar_prefetch=3,
        grid=(N, F),
        in_specs=[pl.BlockSpec(memory_space=pltpu.MemorySpace.HBM)],
        out_specs=pl.BlockSpec(
            (1, 1, H, W),
            lambda n, f, cs_r, kys_r, kxs_r: (n, f, 0, 0),
        ),
        scratch_shapes=[
            pltpu.VMEM((2, C, rows, cols), jnp.float32),
            pltpu.SemaphoreType.DMA((2,)),
        ],
    )

    return pl.pallas_call(
        body,
        grid_spec=grid_spec,
        out_shape=jax.ShapeDtypeStruct((N, F, H, W), jnp.float32),
        compiler_params=pltpu.CompilerParams(
            dimension_semantics=("arbitrary", "arbitrary"),
        ),
    )(cs, kys, kxs, input)
